# baseline (speedup 1.0000x reference)
import functools

import jax
import jax.numpy as jnp
from jax import lax
from jax.experimental import pallas as pl
from jax.experimental.pallas import tpu as pltpu
from jax.experimental.pallas import tpu_sc as plsc

SIGNAL = 8192
DIM = 768
BATCH = 4

NC = 2
NS = 16
L = 16
NW = NC * NS
ROWS_PER_W = SIGNAL // NW
CHUNK = 32
NCHUNK = ROWS_PER_W // CHUNK
CW = CHUNK * DIM
NVEC = CW // L

_mesh = plsc.VectorSubcoreMesh(core_axis_name="c", subcore_axis_name="s")


@functools.partial(
    pl.kernel,
    mesh=_mesh,
    out_type=jax.ShapeDtypeStruct((BATCH, SIGNAL * DIM), jnp.float32),
    scratch_types=[
        pltpu.VMEM((CW,), jnp.float32),
        pltpu.VMEM((CW,), jnp.float32),
        pltpu.SemaphoreType.DMA,
    ],
)
def _pos_add(patches_hbm, pos_hbm, out_hbm, pos_v, buf_v, sem):
    wid = lax.axis_index("s") * NC + lax.axis_index("c")
    base_w = wid * ROWS_PER_W * DIM

    def add_body(i, _):
        sl = pl.ds(i * L, L)
        buf_v[sl] = buf_v[sl] + pos_v[sl]
        return 0

    for c in range(NCHUNK):
        off = base_w + c * CW
        pltpu.sync_copy(pos_hbm.at[pl.ds(off, CW)], pos_v)
        for b in range(BATCH):
            pltpu.sync_copy(patches_hbm.at[b, pl.ds(off, CW)], buf_v)
            lax.fori_loop(0, NVEC, add_body, 0)
            pltpu.sync_copy(buf_v, out_hbm.at[b, pl.ds(off, CW)])


def kernel(patches, pos_table):
    patches_flat = patches.reshape(BATCH, SIGNAL * DIM)
    pos_flat = pos_table.reshape(SIGNAL * DIM)
    out = _pos_add(patches_flat, pos_flat)
    return out.reshape(BATCH, SIGNAL, DIM)


# speedup vs baseline: 3.9847x; 3.9847x over previous
import functools

import jax
import jax.numpy as jnp
from jax.experimental import pallas as pl
from jax.experimental.pallas import tpu as pltpu

SIGNAL = 8192
DIM = 768
BATCH = 4
BS = 256


def _body(patches_ref, pos_ref, out_ref):
    out_ref[0] = patches_ref[0] + pos_ref[...]


def kernel(patches, pos_table):
    grid = (SIGNAL // BS, BATCH)
    return pl.pallas_call(
        _body,
        grid=grid,
        in_specs=[
            pl.BlockSpec((1, BS, DIM), lambda i, b: (b, i, 0)),
            pl.BlockSpec((BS, DIM), lambda i, b: (i, 0)),
        ],
        out_specs=pl.BlockSpec((1, BS, DIM), lambda i, b: (b, i, 0)),
        out_shape=jax.ShapeDtypeStruct((BATCH, SIGNAL, DIM), jnp.float32),
    )(patches, pos_table)
